# baseline (speedup 1.0000x reference)
import functools

import jax
import jax.numpy as jnp
from jax import lax
from jax.experimental import pallas as pl
from jax.experimental.pallas import tpu as pltpu

BW = 16
S = 160
NEGF = float("-inf")
BIGI = 2**31 - 1


def _k1_body(pt_ref, prev_ref, out_ref, *, PB):
    x = pt_ref[...]
    m = jnp.max(x.reshape(PB, S, x.shape[1]), axis=1)
    out_ref[...] = (jnp.log(m) + prev_ref[...])[None]


def _k2a_body(s_ref, prevT_ref, row_ref, col_ref, ps_ref, ba_ref, *, B, P, N):
    s = s_ref[...]
    wio = lax.broadcasted_iota(jnp.int32, (B, BW, P), 1)
    jio = lax.broadcasted_iota(jnp.int32, (B, BW, P), 2)
    pio = wio * P + jio
    prevT = prevT_ref[...]
    w16 = lax.broadcasted_iota(jnp.int32, (B, BW), 1)
    bcol = lax.broadcasted_iota(jnp.int32, (B, 1), 0)
    ROW = jnp.zeros((B, BW), jnp.int32)
    COL = jnp.zeros((B, BW), jnp.int32)
    PS = jnp.zeros((B, BW), jnp.float32)
    BA = jnp.zeros((B, BW), jnp.int32)
    for k in range(BW):
        t = jnp.max(s, axis=2)
        m = jnp.max(t, axis=1, keepdims=True)
        cand = jnp.where(s == m[:, :, None], pio, BIGI)
        c2 = jnp.min(cand, axis=2)
        pid = jnp.min(c2, axis=1, keepdims=True)
        w = pid // P
        j = pid - w * P
        prevk = jnp.sum(jnp.where(w16 == w, prevT, 0.0), axis=1, keepdims=True)
        colm = w16 == k
        ROW = jnp.where(colm, w * 32 + bcol, ROW)
        COL = jnp.where(colm, j * S, COL)
        PS = jnp.where(colm, prevk, PS)
        BA = jnp.where(colm, w * N + j * S, BA)
        s = jnp.where(pio == pid[:, :, None], NEGF, s)
    row_ref[...] = ROW
    col_ref[...] = COL
    ps_ref[...] = PS
    ba_ref[...] = BA


def _gather_body(rows_sm, cols_sm, pt_any, out_ref, win, sems, *, GPB, NG):
    i = pl.program_id(0)

    def issue(group, buf):
        for t in range(GPB):
            p = group * GPB + t
            r = rows_sm[p]
            c = pl.multiple_of(cols_sm[p], 8)
            c0 = pl.multiple_of((r // 128) * 128, 128)
            pltpu.make_async_copy(
                pt_any.at[pl.ds(c, S), pl.ds(c0, 128)],
                win.at[buf, t], sems.at[buf, t]).start()

    @pl.when(i == 0)
    def _():
        issue(0, 0)

    @pl.when(i + 1 < NG)
    def _():
        issue(i + 1, (i + 1) % 2)

    buf = i % 2
    for t in range(GPB):
        pltpu.make_async_copy(
            pt_any.at[pl.ds(0, S), pl.ds(0, 128)],
            win.at[buf, t], sems.at[buf, t]).wait()
    sio = lax.broadcasted_iota(jnp.int32, (128, S), 0)
    for t in range(GPB):
        p = i * GPB + t
        q = rows_sm[p] % 128
        wt = jnp.transpose(win[buf, t], (1, 0))
        rowv = jnp.sum(jnp.where(sio == q, wt, 0.0), axis=0)
        out_ref[0, t, :] = rowv


def _gather(pt, rows, cols):
    NR = rows.shape[0]
    GPB = 16
    NG = NR // GPB
    return pl.pallas_call(
        functools.partial(_gather_body, GPB=GPB, NG=NG),
        grid_spec=pltpu.PrefetchScalarGridSpec(
            num_scalar_prefetch=2,
            grid=(NG,),
            in_specs=[pl.BlockSpec(memory_space=pl.ANY)],
            out_specs=pl.BlockSpec((1, GPB, S), lambda i, rows, cols: (i, 0, 0)),
            scratch_shapes=[
                pltpu.VMEM((2, GPB, S, 128), jnp.float32),
                pltpu.SemaphoreType.DMA((2, GPB)),
            ],
        ),
        out_shape=jax.ShapeDtypeStruct((NG, GPB, S), jnp.float32),
    )(rows, cols, pt)


def _k2c_body(g_ref, ps_ref, ba_ref, sel_ref, logp_ref, bbi_ref, *, B, N):
    g = g_ref[...]
    s = jnp.log(g) + ps_ref[...][:, :, None]
    cio = lax.broadcasted_iota(jnp.int32, (B, BW, S), 2)
    gflat = ba_ref[...][:, :, None] + cio
    w16 = lax.broadcasted_iota(jnp.int32, (B, BW), 1)
    bcol = lax.broadcasted_iota(jnp.int32, (B, 1), 0)
    SEL = jnp.zeros((B, BW), jnp.int32)
    LOGP = jnp.zeros((B, BW), jnp.float32)
    BBI = jnp.zeros((B, BW), jnp.int32)
    for k in range(BW):
        t = jnp.max(s, axis=2)
        m = jnp.max(t, axis=1, keepdims=True)
        cand = jnp.where(s == m[:, :, None], gflat, BIGI)
        c2 = jnp.min(cand, axis=2)
        wf = jnp.min(c2, axis=1, keepdims=True)
        par = wf // N
        colm = w16 == k
        SEL = jnp.where(colm, wf - par * N, SEL)
        LOGP = jnp.where(colm, m, LOGP)
        BBI = jnp.where(colm, bcol + par * B, BBI)
        s = jnp.where(gflat == wf[:, :, None], NEGF, s)
    sel_ref[...] = SEL
    logp_ref[...] = LOGP
    bbi_ref[...] = BBI


def kernel(probs, log_beam_prob_prev):
    A, N = probs.shape
    B = A // BW
    P = N // S
    PB = 5
    pt = probs.T

    prev1 = log_beam_prob_prev.reshape(1, A)

    smaxc = pl.pallas_call(
        functools.partial(_k1_body, PB=PB),
        grid=(P // PB,),
        in_specs=[
            pl.BlockSpec((PB * S, A), lambda i: (i, 0)),
            pl.BlockSpec((1, A), lambda i: (0, 0)),
        ],
        out_specs=pl.BlockSpec((1, PB, A), lambda i: (i, 0, 0)),
        out_shape=jax.ShapeDtypeStruct((P // PB, PB, A), jnp.float32),
    )(pt, prev1)

    smaxT = smaxc.reshape(P, BW, B).transpose(2, 1, 0)
    prevT = log_beam_prob_prev.reshape(BW, B).T

    row, col, ps, ba = pl.pallas_call(
        functools.partial(_k2a_body, B=B, P=P, N=N),
        out_shape=(
            jax.ShapeDtypeStruct((B, BW), jnp.int32),
            jax.ShapeDtypeStruct((B, BW), jnp.int32),
            jax.ShapeDtypeStruct((B, BW), jnp.float32),
            jax.ShapeDtypeStruct((B, BW), jnp.int32),
        ),
    )(smaxT, prevT)

    g = _gather(pt, row.reshape(-1), col.reshape(-1))

    sel, logp, bbi = pl.pallas_call(
        functools.partial(_k2c_body, B=B, N=N),
        out_shape=(
            jax.ShapeDtypeStruct((B, BW), jnp.int32),
            jax.ShapeDtypeStruct((B, BW), jnp.float32),
            jax.ShapeDtypeStruct((B, BW), jnp.int32),
        ),
    )(g, ps, ba)

    return (sel.T.reshape(-1), logp.T.reshape(-1), bbi.T.reshape(-1))


# speedup vs baseline: 1.2699x; 1.2699x over previous
import functools

import jax
import jax.numpy as jnp
from jax import lax
from jax.experimental import pallas as pl
from jax.experimental.pallas import tpu as pltpu

BW = 16
S = 200
NEGF = float("-inf")
BIGI = 2**31 - 1


def _k1_body(pt_ref, prev_ref, out_ref, *, PB):
    x = pt_ref[...]
    m = jnp.max(x.reshape(PB, S, x.shape[1]), axis=1)
    out_ref[...] = (jnp.log(m) + prev_ref[...])[None]


def _k2a_body(s_ref, prevT_ref, row_ref, col_ref, ps_ref, ba_ref, *, B, P, N):
    s = s_ref[...]
    wio = lax.broadcasted_iota(jnp.int32, (B, BW, P), 1)
    jio = lax.broadcasted_iota(jnp.int32, (B, BW, P), 2)
    pio = wio * P + jio
    prevT = prevT_ref[...]
    w16 = lax.broadcasted_iota(jnp.int32, (B, BW), 1)
    bcol = lax.broadcasted_iota(jnp.int32, (B, 1), 0)
    ROW = jnp.zeros((B, BW), jnp.int32)
    COL = jnp.zeros((B, BW), jnp.int32)
    PS = jnp.zeros((B, BW), jnp.float32)
    BA = jnp.zeros((B, BW), jnp.int32)
    for k in range(BW):
        t = jnp.max(s, axis=2)
        m = jnp.max(t, axis=1, keepdims=True)
        cand = jnp.where(s == m[:, :, None], pio, BIGI)
        c2 = jnp.min(cand, axis=2)
        pid = jnp.min(c2, axis=1, keepdims=True)
        w = pid // P
        j = pid - w * P
        prevk = jnp.sum(jnp.where(w16 == w, prevT, 0.0), axis=1, keepdims=True)
        colm = w16 == k
        ROW = jnp.where(colm, w * 32 + bcol, ROW)
        COL = jnp.where(colm, j * S, COL)
        PS = jnp.where(colm, prevk, PS)
        BA = jnp.where(colm, w * N + j * S, BA)
        s = jnp.where(pio == pid[:, :, None], NEGF, s)
    row_ref[...] = ROW
    col_ref[...] = COL
    ps_ref[...] = PS
    ba_ref[...] = BA


def _gather_body(rows_sm, cols_sm, pt_any, out_ref, win, sems, *, GPB, NG):
    i = pl.program_id(0)

    def issue(group, buf):
        for t in range(GPB):
            p = group * GPB + t
            r = rows_sm[p]
            c = pl.multiple_of(cols_sm[p], 8)
            c0 = pl.multiple_of((r // 128) * 128, 128)
            pltpu.make_async_copy(
                pt_any.at[pl.ds(c, S), pl.ds(c0, 128)],
                win.at[buf, t], sems.at[buf, t]).start()

    @pl.when(i == 0)
    def _():
        issue(0, 0)

    @pl.when(i + 1 < NG)
    def _():
        issue(i + 1, (i + 1) % 2)

    buf = i % 2
    for t in range(GPB):
        pltpu.make_async_copy(
            pt_any.at[pl.ds(0, S), pl.ds(0, 128)],
            win.at[buf, t], sems.at[buf, t]).wait()
    sio = lax.broadcasted_iota(jnp.int32, (128, S), 0)
    for t in range(GPB):
        p = i * GPB + t
        q = rows_sm[p] % 128
        wt = jnp.transpose(win[buf, t], (1, 0))
        rowv = jnp.sum(jnp.where(sio == q, wt, 0.0), axis=0)
        out_ref[0, t, :] = rowv


def _gather(pt, rows, cols):
    NR = rows.shape[0]
    GPB = 16
    NG = NR // GPB
    return pl.pallas_call(
        functools.partial(_gather_body, GPB=GPB, NG=NG),
        grid_spec=pltpu.PrefetchScalarGridSpec(
            num_scalar_prefetch=2,
            grid=(NG,),
            in_specs=[pl.BlockSpec(memory_space=pl.ANY)],
            out_specs=pl.BlockSpec((1, GPB, S), lambda i, rows, cols: (i, 0, 0)),
            scratch_shapes=[
                pltpu.VMEM((2, GPB, S, 128), jnp.float32),
                pltpu.SemaphoreType.DMA((2, GPB)),
            ],
        ),
        out_shape=jax.ShapeDtypeStruct((NG, GPB, S), jnp.float32),
    )(rows, cols, pt)


def _k2c_body(g_ref, ps_ref, ba_ref, sel_ref, logp_ref, bbi_ref, *, B, N):
    g = g_ref[...]
    s = jnp.log(g) + ps_ref[...][:, :, None]
    cio = lax.broadcasted_iota(jnp.int32, (B, BW, S), 2)
    gflat = ba_ref[...][:, :, None] + cio
    w16 = lax.broadcasted_iota(jnp.int32, (B, BW), 1)
    bcol = lax.broadcasted_iota(jnp.int32, (B, 1), 0)
    SEL = jnp.zeros((B, BW), jnp.int32)
    LOGP = jnp.zeros((B, BW), jnp.float32)
    BBI = jnp.zeros((B, BW), jnp.int32)
    for k in range(BW):
        t = jnp.max(s, axis=2)
        m = jnp.max(t, axis=1, keepdims=True)
        cand = jnp.where(s == m[:, :, None], gflat, BIGI)
        c2 = jnp.min(cand, axis=2)
        wf = jnp.min(c2, axis=1, keepdims=True)
        par = wf // N
        colm = w16 == k
        SEL = jnp.where(colm, wf - par * N, SEL)
        LOGP = jnp.where(colm, m, LOGP)
        BBI = jnp.where(colm, bcol + par * B, BBI)
        s = jnp.where(gflat == wf[:, :, None], NEGF, s)
    sel_ref[...] = SEL
    logp_ref[...] = LOGP
    bbi_ref[...] = BBI


def kernel(probs, log_beam_prob_prev):
    A, N = probs.shape
    B = A // BW
    P = N // S
    PB = 10
    pt = probs.T

    prev1 = log_beam_prob_prev.reshape(1, A)

    smaxc = pl.pallas_call(
        functools.partial(_k1_body, PB=PB),
        grid=(P // PB,),
        in_specs=[
            pl.BlockSpec((PB * S, A), lambda i: (i, 0)),
            pl.BlockSpec((1, A), lambda i: (0, 0)),
        ],
        out_specs=pl.BlockSpec((1, PB, A), lambda i: (i, 0, 0)),
        out_shape=jax.ShapeDtypeStruct((P // PB, PB, A), jnp.float32),
    )(pt, prev1)

    smaxT = smaxc.reshape(P, BW, B).transpose(2, 1, 0)
    prevT = log_beam_prob_prev.reshape(BW, B).T

    row, col, ps, ba = pl.pallas_call(
        functools.partial(_k2a_body, B=B, P=P, N=N),
        out_shape=(
            jax.ShapeDtypeStruct((B, BW), jnp.int32),
            jax.ShapeDtypeStruct((B, BW), jnp.int32),
            jax.ShapeDtypeStruct((B, BW), jnp.float32),
            jax.ShapeDtypeStruct((B, BW), jnp.int32),
        ),
    )(smaxT, prevT)

    g = _gather(pt, row.reshape(-1), col.reshape(-1))

    sel, logp, bbi = pl.pallas_call(
        functools.partial(_k2c_body, B=B, N=N),
        out_shape=(
            jax.ShapeDtypeStruct((B, BW), jnp.int32),
            jax.ShapeDtypeStruct((B, BW), jnp.float32),
            jax.ShapeDtypeStruct((B, BW), jnp.int32),
        ),
    )(g, ps, ba)

    return (sel.T.reshape(-1), logp.T.reshape(-1), bbi.T.reshape(-1))


# speedup vs baseline: 1.3254x; 1.0437x over previous
import functools

import jax
import jax.numpy as jnp
from jax import lax
from jax.experimental import pallas as pl
from jax.experimental.pallas import tpu as pltpu

BW = 16
S = 200
NEGF = float("-inf")
BIGI = 2**31 - 1


def _k1_body(pt_ref, prev_ref, out_ref, *, PB):
    x = pt_ref[...]
    m = jnp.max(x.reshape(PB, S, x.shape[1]), axis=1)
    out_ref[...] = (jnp.log(m) + prev_ref[...])[None]


def _k2a_body(s_ref, prevT_ref, row_ref, col_ref, ps_ref, ba_ref, *, B, P, N):
    s = s_ref[...]
    wio = lax.broadcasted_iota(jnp.int32, (B, BW, P), 1)
    jio = lax.broadcasted_iota(jnp.int32, (B, BW, P), 2)
    pio_nf = -(wio * P + jio).astype(jnp.float32)
    prevT = prevT_ref[...]
    w16 = lax.broadcasted_iota(jnp.int32, (B, BW), 1)
    bcol = lax.broadcasted_iota(jnp.int32, (B, 1), 0)
    ROW = jnp.zeros((B, BW), jnp.int32)
    COL = jnp.zeros((B, BW), jnp.int32)
    PS = jnp.zeros((B, BW), jnp.float32)
    BA = jnp.zeros((B, BW), jnp.int32)
    for k in range(BW):
        t = jnp.max(s, axis=2)
        m = jnp.max(t, axis=1, keepdims=True)
        cand = jnp.where(s == m[:, :, None], pio_nf, NEGF)
        c2 = jnp.max(cand, axis=2)
        pidn = jnp.max(c2, axis=1, keepdims=True)
        pid = (-pidn).astype(jnp.int32)
        w = pid // P
        j = pid - w * P
        prevk = jnp.sum(jnp.where(w16 == w, prevT, 0.0), axis=1, keepdims=True)
        colm = w16 == k
        ROW = jnp.where(colm, w * 32 + bcol, ROW)
        COL = jnp.where(colm, j * S, COL)
        PS = jnp.where(colm, prevk, PS)
        BA = jnp.where(colm, w * N + j * S, BA)
        s = jnp.where(pio_nf == pidn[:, :, None], NEGF, s)
    row_ref[...] = ROW
    col_ref[...] = COL
    ps_ref[...] = PS
    ba_ref[...] = BA


def _gather_body(rows_sm, cols_sm, pt_any, out_ref, win, sems, *, GPB, NG):
    i = pl.program_id(0)

    def issue(group, buf):
        for t in range(GPB):
            p = group * GPB + t
            r = rows_sm[p]
            c = pl.multiple_of(cols_sm[p], 8)
            c0 = pl.multiple_of((r // 128) * 128, 128)
            pltpu.make_async_copy(
                pt_any.at[pl.ds(c, S), pl.ds(c0, 128)],
                win.at[buf, t], sems.at[buf, t]).start()

    @pl.when(i == 0)
    def _():
        issue(0, 0)

    @pl.when(i + 1 < NG)
    def _():
        issue(i + 1, (i + 1) % 2)

    buf = i % 2
    for t in range(GPB):
        pltpu.make_async_copy(
            pt_any.at[pl.ds(0, S), pl.ds(0, 128)],
            win.at[buf, t], sems.at[buf, t]).wait()
    sio = lax.broadcasted_iota(jnp.int32, (128, S), 0)
    for t in range(GPB):
        p = i * GPB + t
        q = rows_sm[p] % 128
        wt = jnp.transpose(win[buf, t], (1, 0))
        rowv = jnp.sum(jnp.where(sio == q, wt, 0.0), axis=0)
        out_ref[0, t, :] = rowv


def _gather(pt, rows, cols):
    NR = rows.shape[0]
    GPB = 16
    NG = NR // GPB
    return pl.pallas_call(
        functools.partial(_gather_body, GPB=GPB, NG=NG),
        grid_spec=pltpu.PrefetchScalarGridSpec(
            num_scalar_prefetch=2,
            grid=(NG,),
            in_specs=[pl.BlockSpec(memory_space=pl.ANY)],
            out_specs=pl.BlockSpec((1, GPB, S), lambda i, rows, cols: (i, 0, 0)),
            scratch_shapes=[
                pltpu.VMEM((2, GPB, S, 128), jnp.float32),
                pltpu.SemaphoreType.DMA((2, GPB)),
            ],
        ),
        out_shape=jax.ShapeDtypeStruct((NG, GPB, S), jnp.float32),
    )(rows, cols, pt)


def _k2c_body(g_ref, ps_ref, ba_ref, sel_ref, logp_ref, bbi_ref, *, B, N):
    g = g_ref[...]
    s = jnp.log(g) + ps_ref[...][:, :, None]
    cio = lax.broadcasted_iota(jnp.int32, (B, BW, S), 2)
    gflat_nf = -(ba_ref[...][:, :, None] + cio).astype(jnp.float32)
    w16 = lax.broadcasted_iota(jnp.int32, (B, BW), 1)
    bcol = lax.broadcasted_iota(jnp.int32, (B, 1), 0)
    SEL = jnp.zeros((B, BW), jnp.int32)
    LOGP = jnp.zeros((B, BW), jnp.float32)
    BBI = jnp.zeros((B, BW), jnp.int32)
    for k in range(BW):
        t = jnp.max(s, axis=2)
        m = jnp.max(t, axis=1, keepdims=True)
        cand = jnp.where(s == m[:, :, None], gflat_nf, NEGF)
        c2 = jnp.max(cand, axis=2)
        wfn = jnp.max(c2, axis=1, keepdims=True)
        wf = (-wfn).astype(jnp.int32)
        par = wf // N
        colm = w16 == k
        SEL = jnp.where(colm, wf - par * N, SEL)
        LOGP = jnp.where(colm, m, LOGP)
        BBI = jnp.where(colm, bcol + par * B, BBI)
        s = jnp.where(gflat_nf == wfn[:, :, None], NEGF, s)
    sel_ref[...] = SEL
    logp_ref[...] = LOGP
    bbi_ref[...] = BBI


def kernel(probs, log_beam_prob_prev):
    A, N = probs.shape
    B = A // BW
    P = N // S
    PB = 10
    pt = probs.T

    prev1 = log_beam_prob_prev.reshape(1, A)

    smaxc = pl.pallas_call(
        functools.partial(_k1_body, PB=PB),
        grid=(P // PB,),
        in_specs=[
            pl.BlockSpec((PB * S, A), lambda i: (i, 0)),
            pl.BlockSpec((1, A), lambda i: (0, 0)),
        ],
        out_specs=pl.BlockSpec((1, PB, A), lambda i: (i, 0, 0)),
        out_shape=jax.ShapeDtypeStruct((P // PB, PB, A), jnp.float32),
    )(pt, prev1)

    smaxT = smaxc.reshape(P, BW, B).transpose(2, 1, 0)
    prevT = log_beam_prob_prev.reshape(BW, B).T

    row, col, ps, ba = pl.pallas_call(
        functools.partial(_k2a_body, B=B, P=P, N=N),
        out_shape=(
            jax.ShapeDtypeStruct((B, BW), jnp.int32),
            jax.ShapeDtypeStruct((B, BW), jnp.int32),
            jax.ShapeDtypeStruct((B, BW), jnp.float32),
            jax.ShapeDtypeStruct((B, BW), jnp.int32),
        ),
    )(smaxT, prevT)

    g = _gather(pt, row.reshape(-1), col.reshape(-1))

    sel, logp, bbi = pl.pallas_call(
        functools.partial(_k2c_body, B=B, N=N),
        out_shape=(
            jax.ShapeDtypeStruct((B, BW), jnp.int32),
            jax.ShapeDtypeStruct((B, BW), jnp.float32),
            jax.ShapeDtypeStruct((B, BW), jnp.int32),
        ),
    )(g, ps, ba)

    return (sel.T.reshape(-1), logp.T.reshape(-1), bbi.T.reshape(-1))
